# baseline (speedup 1.0000x reference)
import functools

import jax
import jax.numpy as jnp
from jax.experimental import pallas as pl

_SEG = 128


def _proj_body(x_ref, w_ref, asrc_ref, adst_ref, bs_ref, bd_ref,
               fts_ref, f1_ref, f2_ref, *, heads, d):
    fts = jnp.dot(x_ref[...], w_ref[...], preferred_element_type=jnp.float32)
    f1_ref[...] = jnp.dot(fts, asrc_ref[...],
                          preferred_element_type=jnp.float32) + bs_ref[...]
    f2_ref[...] = jnp.dot(fts, adst_ref[...],
                          preferred_element_type=jnp.float32) + bd_ref[...]
    fts_ref[...] = fts
    ones = jnp.ones((fts.shape[0], 1), jnp.float32)
    for h in range(heads):
        fts_ref[:, h * _SEG + d:h * _SEG + d + 1] = ones


def _project(x, w_cat, a_src_cat, a_dst_cat, b_src_row, b_dst_row,
             heads, d, row_block):
    n, fin = x.shape
    dtot = w_cat.shape[1]
    hh = a_src_cat.shape[1]
    grid = (n // row_block,)
    body = functools.partial(_proj_body, heads=heads, d=d)
    return pl.pallas_call(
        body,
        grid=grid,
        in_specs=[
            pl.BlockSpec((row_block, fin), lambda i: (i, 0)),
            pl.BlockSpec((fin, dtot), lambda i: (0, 0)),
            pl.BlockSpec((dtot, hh), lambda i: (0, 0)),
            pl.BlockSpec((dtot, hh), lambda i: (0, 0)),
            pl.BlockSpec((1, hh), lambda i: (0, 0)),
            pl.BlockSpec((1, hh), lambda i: (0, 0)),
        ],
        out_specs=[
            pl.BlockSpec((row_block, dtot), lambda i: (i, 0)),
            pl.BlockSpec((row_block, hh), lambda i: (i, 0)),
            pl.BlockSpec((row_block, hh), lambda i: (i, 0)),
        ],
        out_shape=[
            jax.ShapeDtypeStruct((n, dtot), jnp.float32),
            jax.ShapeDtypeStruct((n, hh), jnp.float32),
            jax.ShapeDtypeStruct((n, hh), jnp.float32),
        ],
    )(x, w_cat, a_src_cat, a_dst_cat, b_src_row, b_dst_row)


def _attn_body(bias_ref, f1_ref, f2t_ref, fts_ref, bout_ref, out_ref,
               *maybe_mask, heads, d, elu, from_mask):
    if from_mask:
        eb = bias_ref[...].astype(jnp.float32)
    else:
        eb = jnp.exp(bias_ref[...])
        maybe_mask[0][...] = eb.astype(jnp.int8)
    for h in range(heads):
        f1 = f1_ref[:, h][:, None]
        f2 = f2t_ref[h, :][None, :]
        u, up = jnp.exp(f1), jnp.exp(0.2 * f1)
        v, vp = jnp.exp(f2), jnp.exp(0.2 * f2)
        e = jnp.maximum(u * v, up * vp) * eb
        num = jnp.dot(e, fts_ref[:, h * _SEG:(h + 1) * _SEG],
                      preferred_element_type=jnp.float32)
        o = num[:, :d] / num[:, d:d + 1] + bout_ref[:, h * d:(h + 1) * d]
        if elu:
            o = jnp.where(o > 0, o, jnp.exp(jnp.minimum(o, 0.0)) - 1.0)
        out_ref[:, h * d:(h + 1) * d] = o


def _attn_layer(bias2d, f1, f2t, fts, b_out_row, heads, d, elu, row_block,
                from_mask=False):
    n = bias2d.shape[0]
    grid = (n // row_block,)
    body = functools.partial(_attn_body, heads=heads, d=d, elu=elu,
                             from_mask=from_mask)
    out_specs = [pl.BlockSpec((row_block, heads * d), lambda i: (i, 0)),
                 pl.BlockSpec((row_block, n), lambda i: (i, 0))]
    out_shape = [jax.ShapeDtypeStruct((n, heads * d), jnp.float32),
                 jax.ShapeDtypeStruct((n, n), jnp.int8)]
    if from_mask:
        out_specs, out_shape = out_specs[:1], out_shape[:1]
    res = pl.pallas_call(
        body,
        grid=grid,
        in_specs=[
            pl.BlockSpec((row_block, n), lambda i: (i, 0)),
            pl.BlockSpec((row_block, heads), lambda i: (i, 0)),
            pl.BlockSpec((heads, n), lambda i: (0, 0)),
            pl.BlockSpec((n, heads * _SEG), lambda i: (0, 0)),
            pl.BlockSpec((1, heads * d), lambda i: (0, 0)),
        ],
        out_specs=out_specs,
        out_shape=out_shape,
    )(bias2d, f1, f2t, fts, b_out_row)
    return (res[0], None) if from_mask else (res[0], res[1])


def _pad_params(W_heads, a_src_heads, a_dst_heads, d):
    heads, fin, _ = W_heads.shape
    w_cat = jnp.zeros((fin, heads * _SEG), jnp.float32)
    a_src = jnp.zeros((heads * _SEG, heads), jnp.float32)
    a_dst = jnp.zeros((heads * _SEG, heads), jnp.float32)
    for h in range(heads):
        w_cat = w_cat.at[:, h * _SEG:h * _SEG + d].set(W_heads[h])
        a_src = a_src.at[h * _SEG:h * _SEG + d, h].set(a_src_heads[h, :, 0])
        a_dst = a_dst.at[h * _SEG:h * _SEG + d, h].set(a_dst_heads[h, :, 0])
    return w_cat, a_src, a_dst


def kernel(inputs, bias_mat, training, W1, a_src1, b_src1, a_dst1, b_dst1,
           bias1, W2, a_src2, b_src2, a_dst2, b_dst2, bias2):
    n = inputs.shape[1]
    f_in = inputs.shape[2]
    heads1, _, h_dim = W1.shape
    c_dim = W2.shape[1]

    x = inputs.reshape(n, f_in)
    bias2d = bias_mat.reshape(n, n)
    rb_proj = 2000 if n % 2000 == 0 else n
    rb_attn = 200 if n % 200 == 0 else n

    w1_cat, a_src1_cat, a_dst1_cat = _pad_params(W1, a_src1, a_dst1, h_dim)
    fts1, f1_1, f2_1 = _project(x, w1_cat, a_src1_cat, a_dst1_cat,
                                b_src1.reshape(1, heads1),
                                b_dst1.reshape(1, heads1),
                                heads1, h_dim, rb_proj)
    h1, mask8 = _attn_layer(bias2d, f1_1, f2_1.T, fts1,
                            bias1.reshape(1, heads1 * h_dim),
                            heads1, h_dim, elu=True, row_block=rb_attn)

    w2_cat, a_src2_cat, a_dst2_cat = _pad_params(
        W2[None], a_src2[None], a_dst2[None], c_dim)
    fts2, f1_2, f2_2 = _project(h1, w2_cat, a_src2_cat, a_dst2_cat,
                                b_src2.reshape(1, 1), b_dst2.reshape(1, 1),
                                1, c_dim, rb_proj)
    out, _ = _attn_layer(mask8, f1_2, f2_2.T, fts2, bias2.reshape(1, c_dim),
                         1, c_dim, elu=False, row_block=rb_attn,
                         from_mask=True)
    return out.reshape(1, n, c_dim)


# speedup vs baseline: 1.0885x; 1.0885x over previous
import functools

import jax
import jax.numpy as jnp
from jax.experimental import pallas as pl

_SEG = 128


def _proj_body(x_ref, w_ref, asrc_ref, adst_ref, bs_ref, bd_ref,
               fts_ref, f1_ref, f2_ref, *, heads, d):
    fts = jnp.dot(x_ref[...], w_ref[...], preferred_element_type=jnp.float32)
    f1_ref[...] = jnp.dot(fts, asrc_ref[...],
                          preferred_element_type=jnp.float32) + bs_ref[...]
    f2_ref[...] = jnp.dot(fts, adst_ref[...],
                          preferred_element_type=jnp.float32) + bd_ref[...]
    fts_ref[...] = fts.astype(jnp.bfloat16)
    ones = jnp.ones((fts.shape[0], 1), jnp.bfloat16)
    for h in range(heads):
        fts_ref[:, h * _SEG + d:h * _SEG + d + 1] = ones


def _project(x, w_cat, a_src_cat, a_dst_cat, b_src_row, b_dst_row,
             heads, d, row_block):
    n, fin = x.shape
    dtot = w_cat.shape[1]
    hh = a_src_cat.shape[1]
    grid = (n // row_block,)
    body = functools.partial(_proj_body, heads=heads, d=d)
    return pl.pallas_call(
        body,
        grid=grid,
        in_specs=[
            pl.BlockSpec((row_block, fin), lambda i: (i, 0)),
            pl.BlockSpec((fin, dtot), lambda i: (0, 0)),
            pl.BlockSpec((dtot, hh), lambda i: (0, 0)),
            pl.BlockSpec((dtot, hh), lambda i: (0, 0)),
            pl.BlockSpec((1, hh), lambda i: (0, 0)),
            pl.BlockSpec((1, hh), lambda i: (0, 0)),
        ],
        out_specs=[
            pl.BlockSpec((row_block, dtot), lambda i: (i, 0)),
            pl.BlockSpec((row_block, hh), lambda i: (i, 0)),
            pl.BlockSpec((row_block, hh), lambda i: (i, 0)),
        ],
        out_shape=[
            jax.ShapeDtypeStruct((n, dtot), jnp.bfloat16),
            jax.ShapeDtypeStruct((n, hh), jnp.float32),
            jax.ShapeDtypeStruct((n, hh), jnp.float32),
        ],
    )(x, w_cat, a_src_cat, a_dst_cat, b_src_row, b_dst_row)


def _attn_body(bias_ref, f1_ref, f2t_ref, fts_ref, bout_ref, out_ref,
               *maybe_mask, heads, d, elu, from_mask):
    if from_mask:
        eb = bias_ref[...].astype(jnp.bfloat16)
    else:
        ebf = jnp.exp(bias_ref[...])
        maybe_mask[0][...] = ebf.astype(jnp.int8)
        eb = ebf.astype(jnp.bfloat16)
    for h in range(heads):
        f1 = f1_ref[:, h][:, None]
        f2 = f2t_ref[h, :][None, :]
        u = jnp.exp(f1).astype(jnp.bfloat16)
        up = jnp.exp(0.2 * f1).astype(jnp.bfloat16)
        v = jnp.exp(f2).astype(jnp.bfloat16)
        vp = jnp.exp(0.2 * f2).astype(jnp.bfloat16)
        e = jnp.maximum(u * v, up * vp) * eb
        num = jnp.dot(e, fts_ref[:, h * _SEG:(h + 1) * _SEG],
                      preferred_element_type=jnp.float32)
        o = num[:, :d] / num[:, d:d + 1] + bout_ref[:, h * d:(h + 1) * d]
        if elu:
            o = jnp.where(o > 0, o, jnp.exp(jnp.minimum(o, 0.0)) - 1.0)
        out_ref[:, h * d:(h + 1) * d] = o


def _attn_layer(bias2d, f1, f2t, fts, b_out_row, heads, d, elu, row_block,
                from_mask=False):
    n = bias2d.shape[0]
    grid = (n // row_block,)
    body = functools.partial(_attn_body, heads=heads, d=d, elu=elu,
                             from_mask=from_mask)
    out_specs = [pl.BlockSpec((row_block, heads * d), lambda i: (i, 0)),
                 pl.BlockSpec((row_block, n), lambda i: (i, 0))]
    out_shape = [jax.ShapeDtypeStruct((n, heads * d), jnp.float32),
                 jax.ShapeDtypeStruct((n, n), jnp.int8)]
    if from_mask:
        out_specs, out_shape = out_specs[:1], out_shape[:1]
    res = pl.pallas_call(
        body,
        grid=grid,
        in_specs=[
            pl.BlockSpec((row_block, n), lambda i: (i, 0)),
            pl.BlockSpec((row_block, heads), lambda i: (i, 0)),
            pl.BlockSpec((heads, n), lambda i: (0, 0)),
            pl.BlockSpec((n, heads * _SEG), lambda i: (0, 0)),
            pl.BlockSpec((1, heads * d), lambda i: (0, 0)),
        ],
        out_specs=out_specs,
        out_shape=out_shape,
    )(bias2d, f1, f2t, fts, b_out_row)
    return (res[0], None) if from_mask else (res[0], res[1])


def _pad_params(W_heads, a_src_heads, a_dst_heads, d):
    heads, fin, _ = W_heads.shape
    w_cat = jnp.zeros((fin, heads * _SEG), jnp.float32)
    a_src = jnp.zeros((heads * _SEG, heads), jnp.float32)
    a_dst = jnp.zeros((heads * _SEG, heads), jnp.float32)
    for h in range(heads):
        w_cat = w_cat.at[:, h * _SEG:h * _SEG + d].set(W_heads[h])
        a_src = a_src.at[h * _SEG:h * _SEG + d, h].set(a_src_heads[h, :, 0])
        a_dst = a_dst.at[h * _SEG:h * _SEG + d, h].set(a_dst_heads[h, :, 0])
    return w_cat, a_src, a_dst


def kernel(inputs, bias_mat, training, W1, a_src1, b_src1, a_dst1, b_dst1,
           bias1, W2, a_src2, b_src2, a_dst2, b_dst2, bias2):
    n = inputs.shape[1]
    f_in = inputs.shape[2]
    heads1, _, h_dim = W1.shape
    c_dim = W2.shape[1]

    x = inputs.reshape(n, f_in)
    bias2d = bias_mat.reshape(n, n)
    rb_proj = 2000 if n % 2000 == 0 else n
    rb_attn = 200 if n % 200 == 0 else n

    w1_cat, a_src1_cat, a_dst1_cat = _pad_params(W1, a_src1, a_dst1, h_dim)
    fts1, f1_1, f2_1 = _project(x, w1_cat, a_src1_cat, a_dst1_cat,
                                b_src1.reshape(1, heads1),
                                b_dst1.reshape(1, heads1),
                                heads1, h_dim, rb_proj)
    h1, mask8 = _attn_layer(bias2d, f1_1, f2_1.T, fts1,
                            bias1.reshape(1, heads1 * h_dim),
                            heads1, h_dim, elu=True, row_block=rb_attn)

    w2_cat, a_src2_cat, a_dst2_cat = _pad_params(
        W2[None], a_src2[None], a_dst2[None], c_dim)
    fts2, f1_2, f2_2 = _project(h1, w2_cat, a_src2_cat, a_dst2_cat,
                                b_src2.reshape(1, 1), b_dst2.reshape(1, 1),
                                1, c_dim, rb_proj)
    out, _ = _attn_layer(mask8, f1_2, f2_2.T, fts2, bias2.reshape(1, c_dim),
                         1, c_dim, elu=False, row_block=rb_attn,
                         from_mask=True)
    return out.reshape(1, n, c_dim)


# speedup vs baseline: 1.1153x; 1.0247x over previous
import functools

import jax
import jax.numpy as jnp
from jax.experimental import pallas as pl

_SEG = 128


def _proj_body(x_ref, w_ref, asrc_ref, adst_ref, bs_ref, bd_ref,
               fts_ref, f1_ref, f2_ref, *, heads, d):
    fts = jnp.dot(x_ref[...], w_ref[...], preferred_element_type=jnp.float32)
    f1_ref[...] = jnp.dot(fts, asrc_ref[...],
                          preferred_element_type=jnp.float32) + bs_ref[...]
    f2 = jnp.dot(fts, adst_ref[...],
                 preferred_element_type=jnp.float32) + bd_ref[...]
    f2_ref[...] = f2
    rb = fts.shape[0]
    vp = jnp.exp(0.2 * f2)
    vp_full = jnp.concatenate(
        [jnp.broadcast_to(vp[:, h:h + 1], (rb, _SEG)) for h in range(heads)],
        axis=1)
    fts_ref[...] = (fts * vp_full).astype(jnp.bfloat16)
    for h in range(heads):
        fts_ref[:, h * _SEG + d:h * _SEG + d + 1] = \
            vp[:, h:h + 1].astype(jnp.bfloat16)


def _project(x, w_cat, a_src_cat, a_dst_cat, b_src_row, b_dst_row,
             heads, d, row_block):
    n, fin = x.shape
    dtot = w_cat.shape[1]
    hh = a_src_cat.shape[1]
    grid = (n // row_block,)
    body = functools.partial(_proj_body, heads=heads, d=d)
    return pl.pallas_call(
        body,
        grid=grid,
        in_specs=[
            pl.BlockSpec((row_block, fin), lambda i: (i, 0)),
            pl.BlockSpec((fin, dtot), lambda i: (0, 0)),
            pl.BlockSpec((dtot, hh), lambda i: (0, 0)),
            pl.BlockSpec((dtot, hh), lambda i: (0, 0)),
            pl.BlockSpec((1, hh), lambda i: (0, 0)),
            pl.BlockSpec((1, hh), lambda i: (0, 0)),
        ],
        out_specs=[
            pl.BlockSpec((row_block, dtot), lambda i: (i, 0)),
            pl.BlockSpec((row_block, hh), lambda i: (i, 0)),
            pl.BlockSpec((row_block, hh), lambda i: (i, 0)),
        ],
        out_shape=[
            jax.ShapeDtypeStruct((n, dtot), jnp.bfloat16),
            jax.ShapeDtypeStruct((n, hh), jnp.float32),
            jax.ShapeDtypeStruct((n, hh), jnp.float32),
        ],
    )(x, w_cat, a_src_cat, a_dst_cat, b_src_row, b_dst_row)


def _attn_body(bias_ref, f1_ref, f2t_ref, fts_ref, bout_ref, out_ref,
               *maybe_mask, heads, d, elu, from_mask):
    if from_mask:
        eb = bias_ref[...].astype(jnp.bfloat16)
    else:
        ebf = jnp.exp(bias_ref[...])
        maybe_mask[0][...] = ebf.astype(jnp.int8)
        eb = ebf.astype(jnp.bfloat16)
    for h in range(heads):
        f1 = f1_ref[:, h][:, None]
        f2 = f2t_ref[h, :][None, :]
        r = jnp.exp(-0.8 * f1).astype(jnp.bfloat16)
        w = jnp.exp(0.8 * f2).astype(jnp.bfloat16)
        e = jnp.maximum(w, r) * eb
        num = jnp.dot(e, fts_ref[:, h * _SEG:(h + 1) * _SEG],
                      preferred_element_type=jnp.float32)
        o = num[:, :d] / num[:, d:d + 1] + bout_ref[:, h * d:(h + 1) * d]
        if elu:
            o = jnp.where(o > 0, o, jnp.exp(jnp.minimum(o, 0.0)) - 1.0)
        out_ref[:, h * d:(h + 1) * d] = o


def _attn_layer(bias2d, f1, f2t, fts, b_out_row, heads, d, elu, row_block,
                from_mask=False):
    n = bias2d.shape[0]
    grid = (n // row_block,)
    body = functools.partial(_attn_body, heads=heads, d=d, elu=elu,
                             from_mask=from_mask)
    out_specs = [pl.BlockSpec((row_block, heads * d), lambda i: (i, 0)),
                 pl.BlockSpec((row_block, n), lambda i: (i, 0))]
    out_shape = [jax.ShapeDtypeStruct((n, heads * d), jnp.float32),
                 jax.ShapeDtypeStruct((n, n), jnp.int8)]
    if from_mask:
        out_specs, out_shape = out_specs[:1], out_shape[:1]
    res = pl.pallas_call(
        body,
        grid=grid,
        in_specs=[
            pl.BlockSpec((row_block, n), lambda i: (i, 0)),
            pl.BlockSpec((row_block, heads), lambda i: (i, 0)),
            pl.BlockSpec((heads, n), lambda i: (0, 0)),
            pl.BlockSpec((n, heads * _SEG), lambda i: (0, 0)),
            pl.BlockSpec((1, heads * d), lambda i: (0, 0)),
        ],
        out_specs=out_specs,
        out_shape=out_shape,
    )(bias2d, f1, f2t, fts, b_out_row)
    return (res[0], None) if from_mask else (res[0], res[1])


def _pad_params(W_heads, a_src_heads, a_dst_heads, d):
    heads, fin, _ = W_heads.shape
    w_cat = jnp.zeros((fin, heads * _SEG), jnp.float32)
    a_src = jnp.zeros((heads * _SEG, heads), jnp.float32)
    a_dst = jnp.zeros((heads * _SEG, heads), jnp.float32)
    for h in range(heads):
        w_cat = w_cat.at[:, h * _SEG:h * _SEG + d].set(W_heads[h])
        a_src = a_src.at[h * _SEG:h * _SEG + d, h].set(a_src_heads[h, :, 0])
        a_dst = a_dst.at[h * _SEG:h * _SEG + d, h].set(a_dst_heads[h, :, 0])
    return w_cat, a_src, a_dst


def kernel(inputs, bias_mat, training, W1, a_src1, b_src1, a_dst1, b_dst1,
           bias1, W2, a_src2, b_src2, a_dst2, b_dst2, bias2):
    n = inputs.shape[1]
    f_in = inputs.shape[2]
    heads1, _, h_dim = W1.shape
    c_dim = W2.shape[1]

    x = inputs.reshape(n, f_in)
    bias2d = bias_mat.reshape(n, n)
    rb_proj = 2000 if n % 2000 == 0 else n
    rb_attn = 200 if n % 200 == 0 else n

    w1_cat, a_src1_cat, a_dst1_cat = _pad_params(W1, a_src1, a_dst1, h_dim)
    fts1, f1_1, f2_1 = _project(x, w1_cat, a_src1_cat, a_dst1_cat,
                                b_src1.reshape(1, heads1),
                                b_dst1.reshape(1, heads1),
                                heads1, h_dim, rb_proj)
    h1, mask8 = _attn_layer(bias2d, f1_1, f2_1.T, fts1,
                            bias1.reshape(1, heads1 * h_dim),
                            heads1, h_dim, elu=True, row_block=rb_attn)

    w2_cat, a_src2_cat, a_dst2_cat = _pad_params(
        W2[None], a_src2[None], a_dst2[None], c_dim)
    fts2, f1_2, f2_2 = _project(h1, w2_cat, a_src2_cat, a_dst2_cat,
                                b_src2.reshape(1, 1), b_dst2.reshape(1, 1),
                                1, c_dim, rb_proj)
    out, _ = _attn_layer(mask8, f1_2, f2_2.T, fts2, bias2.reshape(1, c_dim),
                         1, c_dim, elu=False, row_block=rb_attn,
                         from_mask=True)
    return out.reshape(1, n, c_dim)


# speedup vs baseline: 1.1570x; 1.0373x over previous
import functools

import jax
import jax.numpy as jnp
from jax.experimental import pallas as pl

_SEG = 128


def _proj_body(x_ref, w_ref, asrc_ref, adst_ref, bs_ref, bd_ref,
               fts_ref, f1_ref, f2_ref, *, heads, d):
    fts = jnp.dot(x_ref[...], w_ref[...], preferred_element_type=jnp.float32)
    f1 = jnp.dot(fts, asrc_ref[...],
                 preferred_element_type=jnp.float32) + bs_ref[...]
    f2 = jnp.dot(fts, adst_ref[...],
                 preferred_element_type=jnp.float32) + bd_ref[...]
    f1_ref[...] = jnp.exp(-0.8 * f1).astype(jnp.bfloat16)
    f2_ref[...] = jnp.exp(0.8 * f2).astype(jnp.bfloat16)
    rb = fts.shape[0]
    vp = jnp.exp(0.2 * f2)
    vp_full = jnp.concatenate(
        [jnp.broadcast_to(vp[:, h:h + 1], (rb, _SEG)) for h in range(heads)],
        axis=1)
    fts_ref[...] = (fts * vp_full).astype(jnp.bfloat16)
    for h in range(heads):
        fts_ref[:, h * _SEG + d:h * _SEG + d + 1] = \
            vp[:, h:h + 1].astype(jnp.bfloat16)


def _project(x, w_cat, a_src_cat, a_dst_cat, b_src_row, b_dst_row,
             heads, d, row_block):
    n, fin = x.shape
    dtot = w_cat.shape[1]
    hh = a_src_cat.shape[1]
    grid = (n // row_block,)
    body = functools.partial(_proj_body, heads=heads, d=d)
    return pl.pallas_call(
        body,
        grid=grid,
        in_specs=[
            pl.BlockSpec((row_block, fin), lambda i: (i, 0)),
            pl.BlockSpec((fin, dtot), lambda i: (0, 0)),
            pl.BlockSpec((dtot, hh), lambda i: (0, 0)),
            pl.BlockSpec((dtot, hh), lambda i: (0, 0)),
            pl.BlockSpec((1, hh), lambda i: (0, 0)),
            pl.BlockSpec((1, hh), lambda i: (0, 0)),
        ],
        out_specs=[
            pl.BlockSpec((row_block, dtot), lambda i: (i, 0)),
            pl.BlockSpec((row_block, hh), lambda i: (i, 0)),
            pl.BlockSpec((row_block, hh), lambda i: (i, 0)),
        ],
        out_shape=[
            jax.ShapeDtypeStruct((n, dtot), jnp.bfloat16),
            jax.ShapeDtypeStruct((n, hh), jnp.bfloat16),
            jax.ShapeDtypeStruct((n, hh), jnp.bfloat16),
        ],
    )(x, w_cat, a_src_cat, a_dst_cat, b_src_row, b_dst_row)


def _attn_body(bias_ref, f1_ref, f2t_ref, fts_ref, bout_ref, out_ref,
               *maybe_mask, heads, d, elu, from_mask):
    if from_mask:
        eb = bias_ref[...].astype(jnp.bfloat16)
    else:
        eb = jnp.exp(bias_ref[...]).astype(jnp.bfloat16)
        maybe_mask[0][...] = eb.astype(jnp.int8)
    for h in range(heads):
        r = f1_ref[:, h][:, None]
        w = f2t_ref[h, :][None, :]
        e = jnp.maximum(w, r) * eb
        num = jnp.dot(e, fts_ref[:, h * _SEG:(h + 1) * _SEG],
                      preferred_element_type=jnp.float32)
        o = num[:, :d] / num[:, d:d + 1] + bout_ref[:, h * d:(h + 1) * d]
        if elu:
            o = jnp.where(o > 0, o, jnp.exp(jnp.minimum(o, 0.0)) - 1.0)
        out_ref[:, h * d:(h + 1) * d] = o


def _attn_layer(bias2d, f1, f2t, fts, b_out_row, heads, d, elu, row_block,
                from_mask=False):
    n = bias2d.shape[0]
    grid = (n // row_block,)
    body = functools.partial(_attn_body, heads=heads, d=d, elu=elu,
                             from_mask=from_mask)
    out_specs = [pl.BlockSpec((row_block, heads * d), lambda i: (i, 0)),
                 pl.BlockSpec((row_block, n), lambda i: (i, 0))]
    out_shape = [jax.ShapeDtypeStruct((n, heads * d), jnp.float32),
                 jax.ShapeDtypeStruct((n, n), jnp.int8)]
    if from_mask:
        out_specs, out_shape = out_specs[:1], out_shape[:1]
    res = pl.pallas_call(
        body,
        grid=grid,
        in_specs=[
            pl.BlockSpec((row_block, n), lambda i: (i, 0)),
            pl.BlockSpec((row_block, heads), lambda i: (i, 0)),
            pl.BlockSpec((heads, n), lambda i: (0, 0)),
            pl.BlockSpec((n, heads * _SEG), lambda i: (0, 0)),
            pl.BlockSpec((1, heads * d), lambda i: (0, 0)),
        ],
        out_specs=out_specs,
        out_shape=out_shape,
    )(bias2d, f1, f2t, fts, b_out_row)
    return (res[0], None) if from_mask else (res[0], res[1])


def _pad_params(W_heads, a_src_heads, a_dst_heads, d):
    heads, fin, _ = W_heads.shape
    w_cat = jnp.zeros((fin, heads * _SEG), jnp.float32)
    a_src = jnp.zeros((heads * _SEG, heads), jnp.float32)
    a_dst = jnp.zeros((heads * _SEG, heads), jnp.float32)
    for h in range(heads):
        w_cat = w_cat.at[:, h * _SEG:h * _SEG + d].set(W_heads[h])
        a_src = a_src.at[h * _SEG:h * _SEG + d, h].set(a_src_heads[h, :, 0])
        a_dst = a_dst.at[h * _SEG:h * _SEG + d, h].set(a_dst_heads[h, :, 0])
    return w_cat, a_src, a_dst


def kernel(inputs, bias_mat, training, W1, a_src1, b_src1, a_dst1, b_dst1,
           bias1, W2, a_src2, b_src2, a_dst2, b_dst2, bias2):
    n = inputs.shape[1]
    f_in = inputs.shape[2]
    heads1, _, h_dim = W1.shape
    c_dim = W2.shape[1]

    x = inputs.reshape(n, f_in)
    bias2d = bias_mat.reshape(n, n)
    rb_proj = 2000 if n % 2000 == 0 else n
    rb_attn = 200 if n % 200 == 0 else n

    w1_cat, a_src1_cat, a_dst1_cat = _pad_params(W1, a_src1, a_dst1, h_dim)
    fts1, f1_1, f2_1 = _project(x, w1_cat, a_src1_cat, a_dst1_cat,
                                b_src1.reshape(1, heads1),
                                b_dst1.reshape(1, heads1),
                                heads1, h_dim, rb_proj)
    h1, mask8 = _attn_layer(bias2d, f1_1, f2_1.T, fts1,
                            bias1.reshape(1, heads1 * h_dim),
                            heads1, h_dim, elu=True, row_block=rb_attn)

    w2_cat, a_src2_cat, a_dst2_cat = _pad_params(
        W2[None], a_src2[None], a_dst2[None], c_dim)
    fts2, f1_2, f2_2 = _project(h1, w2_cat, a_src2_cat, a_dst2_cat,
                                b_src2.reshape(1, 1), b_dst2.reshape(1, 1),
                                1, c_dim, rb_proj)
    out, _ = _attn_layer(mask8, f1_2, f2_2.T, fts2, bias2.reshape(1, c_dim),
                         1, c_dim, elu=False, row_block=rb_attn,
                         from_mask=True)
    return out.reshape(1, n, c_dim)


# speedup vs baseline: 1.1830x; 1.0225x over previous
import functools

import jax
import jax.numpy as jnp
from jax.experimental import pallas as pl

_SEG = 128


def _proj_body(x_ref, w_ref, asrc_ref, adst_ref, bs_ref, bd_ref,
               fts_ref, f1_ref, f2_ref, *, heads, d):
    fts = jnp.dot(x_ref[...], w_ref[...], preferred_element_type=jnp.float32)
    f1 = jnp.dot(fts, asrc_ref[...],
                 preferred_element_type=jnp.float32) + bs_ref[...]
    f2 = jnp.dot(fts, adst_ref[...],
                 preferred_element_type=jnp.float32) + bd_ref[...]
    f1_ref[...] = jnp.exp(-0.8 * f1).astype(jnp.bfloat16)
    f2_ref[...] = jnp.exp(0.8 * f2).astype(jnp.bfloat16)
    rb = fts.shape[0]
    vp = jnp.exp(0.2 * f2)
    vp_full = jnp.concatenate(
        [jnp.broadcast_to(vp[:, h:h + 1], (rb, _SEG)) for h in range(heads)],
        axis=1)
    fts_ref[...] = (fts * vp_full).astype(jnp.bfloat16)
    for h in range(heads):
        fts_ref[:, h * _SEG + d:h * _SEG + d + 1] = \
            vp[:, h:h + 1].astype(jnp.bfloat16)


def _project(x, w_cat, a_src_cat, a_dst_cat, b_src_row, b_dst_row,
             heads, d, row_block):
    n, fin = x.shape
    dtot = w_cat.shape[1]
    hh = a_src_cat.shape[1]
    grid = (n // row_block,)
    body = functools.partial(_proj_body, heads=heads, d=d)
    return pl.pallas_call(
        body,
        grid=grid,
        in_specs=[
            pl.BlockSpec((row_block, fin), lambda i: (i, 0)),
            pl.BlockSpec((fin, dtot), lambda i: (0, 0)),
            pl.BlockSpec((dtot, hh), lambda i: (0, 0)),
            pl.BlockSpec((dtot, hh), lambda i: (0, 0)),
            pl.BlockSpec((1, hh), lambda i: (0, 0)),
            pl.BlockSpec((1, hh), lambda i: (0, 0)),
        ],
        out_specs=[
            pl.BlockSpec((row_block, dtot), lambda i: (i, 0)),
            pl.BlockSpec((row_block, hh), lambda i: (i, 0)),
            pl.BlockSpec((row_block, hh), lambda i: (i, 0)),
        ],
        out_shape=[
            jax.ShapeDtypeStruct((n, dtot), jnp.bfloat16),
            jax.ShapeDtypeStruct((n, hh), jnp.bfloat16),
            jax.ShapeDtypeStruct((n, hh), jnp.bfloat16),
        ],
    )(x, w_cat, a_src_cat, a_dst_cat, b_src_row, b_dst_row)


def _attn_body(bias_ref, f1_ref, f2t_ref, fts_ref, bout_ref, out_ref,
               *maybe_mask, heads, d, elu, from_mask):
    if from_mask:
        eb = bias_ref[...].astype(jnp.bfloat16)
    else:
        bias_bf = bias_ref[...].astype(jnp.bfloat16)
        eb = jnp.clip(bias_bf + 1.0, 0.0, 1.0)
        maybe_mask[0][...] = eb.astype(jnp.int8)
    for h in range(heads):
        r = f1_ref[:, h][:, None]
        w = f2t_ref[h, :][None, :]
        e = jnp.maximum(w, r) * eb
        num = jnp.dot(e, fts_ref[:, h * _SEG:(h + 1) * _SEG],
                      preferred_element_type=jnp.float32)
        o = num[:, :d] / num[:, d:d + 1] + bout_ref[:, h * d:(h + 1) * d]
        if elu:
            o = jnp.where(o > 0, o, jnp.exp(jnp.minimum(o, 0.0)) - 1.0)
        out_ref[:, h * d:(h + 1) * d] = o


def _attn_layer(bias2d, f1, f2t, fts, b_out_row, heads, d, elu, row_block,
                from_mask=False):
    n = bias2d.shape[0]
    grid = (n // row_block,)
    body = functools.partial(_attn_body, heads=heads, d=d, elu=elu,
                             from_mask=from_mask)
    out_specs = [pl.BlockSpec((row_block, heads * d), lambda i: (i, 0)),
                 pl.BlockSpec((row_block, n), lambda i: (i, 0))]
    out_shape = [jax.ShapeDtypeStruct((n, heads * d), jnp.float32),
                 jax.ShapeDtypeStruct((n, n), jnp.int8)]
    if from_mask:
        out_specs, out_shape = out_specs[:1], out_shape[:1]
    res = pl.pallas_call(
        body,
        grid=grid,
        in_specs=[
            pl.BlockSpec((row_block, n), lambda i: (i, 0)),
            pl.BlockSpec((row_block, heads), lambda i: (i, 0)),
            pl.BlockSpec((heads, n), lambda i: (0, 0)),
            pl.BlockSpec((n, heads * _SEG), lambda i: (0, 0)),
            pl.BlockSpec((1, heads * d), lambda i: (0, 0)),
        ],
        out_specs=out_specs,
        out_shape=out_shape,
    )(bias2d, f1, f2t, fts, b_out_row)
    return (res[0], None) if from_mask else (res[0], res[1])


def _pad_params(W_heads, a_src_heads, a_dst_heads, d):
    heads, fin, _ = W_heads.shape
    w_cat = jnp.zeros((fin, heads * _SEG), jnp.float32)
    a_src = jnp.zeros((heads * _SEG, heads), jnp.float32)
    a_dst = jnp.zeros((heads * _SEG, heads), jnp.float32)
    for h in range(heads):
        w_cat = w_cat.at[:, h * _SEG:h * _SEG + d].set(W_heads[h])
        a_src = a_src.at[h * _SEG:h * _SEG + d, h].set(a_src_heads[h, :, 0])
        a_dst = a_dst.at[h * _SEG:h * _SEG + d, h].set(a_dst_heads[h, :, 0])
    return w_cat, a_src, a_dst


def kernel(inputs, bias_mat, training, W1, a_src1, b_src1, a_dst1, b_dst1,
           bias1, W2, a_src2, b_src2, a_dst2, b_dst2, bias2):
    n = inputs.shape[1]
    f_in = inputs.shape[2]
    heads1, _, h_dim = W1.shape
    c_dim = W2.shape[1]

    x = inputs.reshape(n, f_in)
    bias2d = bias_mat.reshape(n, n)
    rb_proj = 2000 if n % 2000 == 0 else n
    rb_attn = 200 if n % 200 == 0 else n

    w1_cat, a_src1_cat, a_dst1_cat = _pad_params(W1, a_src1, a_dst1, h_dim)
    fts1, f1_1, f2_1 = _project(x, w1_cat, a_src1_cat, a_dst1_cat,
                                b_src1.reshape(1, heads1),
                                b_dst1.reshape(1, heads1),
                                heads1, h_dim, rb_proj)
    h1, mask8 = _attn_layer(bias2d, f1_1, f2_1.T, fts1,
                            bias1.reshape(1, heads1 * h_dim),
                            heads1, h_dim, elu=True, row_block=rb_attn)

    w2_cat, a_src2_cat, a_dst2_cat = _pad_params(
        W2[None], a_src2[None], a_dst2[None], c_dim)
    fts2, f1_2, f2_2 = _project(h1, w2_cat, a_src2_cat, a_dst2_cat,
                                b_src2.reshape(1, 1), b_dst2.reshape(1, 1),
                                1, c_dim, rb_proj)
    out, _ = _attn_layer(mask8, f1_2, f2_2.T, fts2, bias2.reshape(1, c_dim),
                         1, c_dim, elu=False, row_block=rb_attn,
                         from_mask=True)
    return out.reshape(1, n, c_dim)


# speedup vs baseline: 1.2408x; 1.0488x over previous
import functools

import jax
import jax.numpy as jnp
from jax.experimental import pallas as pl

_SEG = 128


def _proj_body(x_ref, w_ref, asrc_ref, adst_ref, bs_ref, bd_ref,
               fts_ref, f1_ref, f2_ref, *, heads, d):
    fts = jnp.dot(x_ref[...], w_ref[...], preferred_element_type=jnp.float32)
    f1 = jnp.dot(fts, asrc_ref[...],
                 preferred_element_type=jnp.float32) + bs_ref[...]
    f2 = jnp.dot(fts, adst_ref[...],
                 preferred_element_type=jnp.float32) + bd_ref[...]
    f1_ref[...] = jnp.exp(-0.8 * f1).astype(jnp.bfloat16)
    f2_ref[...] = jnp.exp(0.8 * f2).astype(jnp.bfloat16)
    rb = fts.shape[0]
    vp = jnp.exp(0.2 * f2)
    vp_full = jnp.concatenate(
        [jnp.broadcast_to(vp[:, h:h + 1], (rb, _SEG)) for h in range(heads)],
        axis=1)
    fts_ref[...] = (fts * vp_full).astype(jnp.bfloat16)
    for h in range(heads):
        fts_ref[:, h * _SEG + d:h * _SEG + d + 1] = \
            vp[:, h:h + 1].astype(jnp.bfloat16)


def _project(x, w_cat, a_src_cat, a_dst_cat, b_src_row, b_dst_row,
             heads, d, row_block):
    n, fin = x.shape
    dtot = w_cat.shape[1]
    hh = a_src_cat.shape[1]
    grid = (n // row_block,)
    body = functools.partial(_proj_body, heads=heads, d=d)
    return pl.pallas_call(
        body,
        grid=grid,
        in_specs=[
            pl.BlockSpec((row_block, fin), lambda i: (i, 0)),
            pl.BlockSpec((fin, dtot), lambda i: (0, 0)),
            pl.BlockSpec((dtot, hh), lambda i: (0, 0)),
            pl.BlockSpec((dtot, hh), lambda i: (0, 0)),
            pl.BlockSpec((1, hh), lambda i: (0, 0)),
            pl.BlockSpec((1, hh), lambda i: (0, 0)),
        ],
        out_specs=[
            pl.BlockSpec((row_block, dtot), lambda i: (i, 0)),
            pl.BlockSpec((row_block, hh), lambda i: (i, 0)),
            pl.BlockSpec((row_block, hh), lambda i: (i, 0)),
        ],
        out_shape=[
            jax.ShapeDtypeStruct((n, dtot), jnp.bfloat16),
            jax.ShapeDtypeStruct((n, hh), jnp.bfloat16),
            jax.ShapeDtypeStruct((n, hh), jnp.bfloat16),
        ],
    )(x, w_cat, a_src_cat, a_dst_cat, b_src_row, b_dst_row)


def _attn_body(bias_ref, f1_ref, f2t_ref, fts_ref, bout_ref, out_ref,
               *maybe_mask, heads, d, elu, from_mask):
    if from_mask:
        eb = bias_ref[...].astype(jnp.bfloat16)
    else:
        bias_bf = bias_ref[...].astype(jnp.bfloat16)
        eb = jnp.clip(bias_bf + 1.0, 0.0, 1.0)
        maybe_mask[0][...] = eb.astype(jnp.int8)
    for h in range(heads):
        r = f1_ref[:, h][:, None]
        w = f2t_ref[h, :][None, :]
        e = jnp.maximum(w, r) * eb
        num = jnp.dot(e, fts_ref[:, h * _SEG:(h + 1) * _SEG],
                      preferred_element_type=jnp.float32)
        o = num[:, :d] / num[:, d:d + 1] + bout_ref[:, h * d:(h + 1) * d]
        if elu:
            o = jnp.where(o > 0, o, jnp.exp(jnp.minimum(o, 0.0)) - 1.0)
        out_ref[:, h * d:(h + 1) * d] = o


def _attn_layer(bias2d, f1, f2t, fts, b_out_row, heads, d, elu, row_block,
                from_mask=False):
    n = bias2d.shape[0]
    grid = (n // row_block,)
    body = functools.partial(_attn_body, heads=heads, d=d, elu=elu,
                             from_mask=from_mask)
    out_specs = [pl.BlockSpec((row_block, heads * d), lambda i: (i, 0)),
                 pl.BlockSpec((row_block, n), lambda i: (i, 0))]
    out_shape = [jax.ShapeDtypeStruct((n, heads * d), jnp.float32),
                 jax.ShapeDtypeStruct((n, n), jnp.int8)]
    if from_mask:
        out_specs, out_shape = out_specs[:1], out_shape[:1]
    res = pl.pallas_call(
        body,
        grid=grid,
        in_specs=[
            pl.BlockSpec((row_block, n), lambda i: (i, 0)),
            pl.BlockSpec((row_block, heads), lambda i: (i, 0)),
            pl.BlockSpec((heads, n), lambda i: (0, 0)),
            pl.BlockSpec((n, heads * _SEG), lambda i: (0, 0)),
            pl.BlockSpec((1, heads * d), lambda i: (0, 0)),
        ],
        out_specs=out_specs,
        out_shape=out_shape,
    )(bias2d, f1, f2t, fts, b_out_row)
    return (res[0], None) if from_mask else (res[0], res[1])


def _pad_params(W_heads, a_src_heads, a_dst_heads, d):
    heads, fin, _ = W_heads.shape
    w_cat = jnp.zeros((fin, heads * _SEG), jnp.float32)
    a_src = jnp.zeros((heads * _SEG, heads), jnp.float32)
    a_dst = jnp.zeros((heads * _SEG, heads), jnp.float32)
    for h in range(heads):
        w_cat = w_cat.at[:, h * _SEG:h * _SEG + d].set(W_heads[h])
        a_src = a_src.at[h * _SEG:h * _SEG + d, h].set(a_src_heads[h, :, 0])
        a_dst = a_dst.at[h * _SEG:h * _SEG + d, h].set(a_dst_heads[h, :, 0])
    return w_cat, a_src, a_dst


def kernel(inputs, bias_mat, training, W1, a_src1, b_src1, a_dst1, b_dst1,
           bias1, W2, a_src2, b_src2, a_dst2, b_dst2, bias2):
    n = inputs.shape[1]
    f_in = inputs.shape[2]
    heads1, _, h_dim = W1.shape
    c_dim = W2.shape[1]

    x = inputs.reshape(n, f_in)
    bias2d = bias_mat.reshape(n, n)
    rb_proj = 2000 if n % 2000 == 0 else n
    rb_attn = 200 if n % 200 == 0 else n

    w1_cat, a_src1_cat, a_dst1_cat = _pad_params(W1, a_src1, a_dst1, h_dim)
    fts1, f1_1, f2_1 = _project(x, w1_cat, a_src1_cat, a_dst1_cat,
                                b_src1.reshape(1, heads1),
                                b_dst1.reshape(1, heads1),
                                heads1, h_dim, rb_proj)
    h1, mask8 = _attn_layer(bias2d, f1_1, f2_1.T, fts1,
                            bias1.reshape(1, heads1 * h_dim),
                            heads1, h_dim, elu=True, row_block=rb_attn)

    w2_cat, a_src2_cat, a_dst2_cat = _pad_params(
        W2[None], a_src2[None], a_dst2[None], c_dim)
    fts2, f1_2, f2_2 = _project(h1, w2_cat, a_src2_cat, a_dst2_cat,
                                b_src2.reshape(1, 1), b_dst2.reshape(1, 1),
                                1, c_dim, rb_proj)
    rb_attn2 = 400 if n % 400 == 0 else rb_attn
    out, _ = _attn_layer(mask8, f1_2, f2_2.T, fts2, bias2.reshape(1, c_dim),
                         1, c_dim, elu=False, row_block=rb_attn2,
                         from_mask=True)
    return out.reshape(1, n, c_dim)


# speedup vs baseline: 1.3352x; 1.0761x over previous
import functools

import jax
import jax.numpy as jnp
from jax.experimental import pallas as pl

_SEG = 128


def _proj_body(x_ref, w_ref, asrc_ref, adst_ref, bs_ref, bd_ref,
               fts_ref, f1_ref, f2_ref, *, heads, d):
    fts = jnp.dot(x_ref[...], w_ref[...], preferred_element_type=jnp.float32)
    f1 = jnp.dot(fts, asrc_ref[...],
                 preferred_element_type=jnp.float32) + bs_ref[...]
    f2 = jnp.dot(fts, adst_ref[...],
                 preferred_element_type=jnp.float32) + bd_ref[...]
    f1_ref[...] = jnp.exp(-0.8 * f1).astype(jnp.bfloat16)
    f2_ref[...] = jnp.exp(0.8 * f2).astype(jnp.bfloat16)
    rb = fts.shape[0]
    vp = jnp.exp(0.2 * f2)
    vp_full = jnp.concatenate(
        [jnp.broadcast_to(vp[:, h:h + 1], (rb, _SEG)) for h in range(heads)],
        axis=1)
    fts_ref[...] = (fts * vp_full).astype(jnp.bfloat16)
    for h in range(heads):
        fts_ref[:, h * _SEG + d:h * _SEG + d + 1] = \
            vp[:, h:h + 1].astype(jnp.bfloat16)


def _project(x, w_cat, a_src_cat, a_dst_cat, b_src_row, b_dst_row,
             heads, d, row_block):
    n, fin = x.shape
    dtot = w_cat.shape[1]
    hh = a_src_cat.shape[1]
    grid = (n // row_block,)
    body = functools.partial(_proj_body, heads=heads, d=d)
    return pl.pallas_call(
        body,
        grid=grid,
        in_specs=[
            pl.BlockSpec((row_block, fin), lambda i: (i, 0)),
            pl.BlockSpec((fin, dtot), lambda i: (0, 0)),
            pl.BlockSpec((dtot, hh), lambda i: (0, 0)),
            pl.BlockSpec((dtot, hh), lambda i: (0, 0)),
            pl.BlockSpec((1, hh), lambda i: (0, 0)),
            pl.BlockSpec((1, hh), lambda i: (0, 0)),
        ],
        out_specs=[
            pl.BlockSpec((row_block, dtot), lambda i: (i, 0)),
            pl.BlockSpec((row_block, hh), lambda i: (i, 0)),
            pl.BlockSpec((row_block, hh), lambda i: (i, 0)),
        ],
        out_shape=[
            jax.ShapeDtypeStruct((n, dtot), jnp.bfloat16),
            jax.ShapeDtypeStruct((n, hh), jnp.bfloat16),
            jax.ShapeDtypeStruct((n, hh), jnp.bfloat16),
        ],
    )(x, w_cat, a_src_cat, a_dst_cat, b_src_row, b_dst_row)


def _attn_body(bias_ref, f1_ref, f2t_ref, fts_ref, bout_ref, out_ref,
               *maybe_mask, heads, d, elu, from_mask):
    if from_mask:
        eb = bias_ref[...].astype(jnp.bfloat16)
    else:
        bias_bf = bias_ref[...].astype(jnp.bfloat16)
        eb = jnp.clip(bias_bf + 1.0, 0.0, 1.0)
        maybe_mask[0][...] = eb.astype(jnp.int8)
    for h in range(heads):
        r = f1_ref[:, h][:, None]
        w = f2t_ref[h, :][None, :]
        e = jnp.maximum(w, r) * eb
        num = jnp.dot(e, fts_ref[:, h * _SEG:(h + 1) * _SEG],
                      preferred_element_type=jnp.float32)
        o = num[:, :d] / num[:, d:d + 1] + bout_ref[:, h * d:(h + 1) * d]
        if elu:
            o = jnp.where(o > 0, o, jnp.exp(jnp.minimum(o, 0.0)) - 1.0)
        out_ref[:, h * d:(h + 1) * d] = o


def _attn_layer(bias2d, f1, f2t, fts, b_out_row, heads, d, elu, row_block,
                from_mask=False):
    n = bias2d.shape[0]
    grid = (n // row_block,)
    body = functools.partial(_attn_body, heads=heads, d=d, elu=elu,
                             from_mask=from_mask)
    out_specs = [pl.BlockSpec((row_block, heads * d), lambda i: (i, 0)),
                 pl.BlockSpec((row_block, n), lambda i: (i, 0))]
    out_shape = [jax.ShapeDtypeStruct((n, heads * d), jnp.float32),
                 jax.ShapeDtypeStruct((n, n), jnp.int8)]
    if from_mask:
        out_specs, out_shape = out_specs[:1], out_shape[:1]
    res = pl.pallas_call(
        body,
        grid=grid,
        in_specs=[
            pl.BlockSpec((row_block, n), lambda i: (i, 0)),
            pl.BlockSpec((row_block, heads), lambda i: (i, 0)),
            pl.BlockSpec((heads, n), lambda i: (0, 0)),
            pl.BlockSpec((n, heads * _SEG), lambda i: (0, 0)),
            pl.BlockSpec((1, heads * d), lambda i: (0, 0)),
        ],
        out_specs=out_specs,
        out_shape=out_shape,
    )(bias2d, f1, f2t, fts, b_out_row)
    return (res[0], None) if from_mask else (res[0], res[1])


def _pad_params(W_heads, a_src_heads, a_dst_heads, d):
    heads, fin, _ = W_heads.shape
    w_cat = jnp.zeros((fin, heads * _SEG), jnp.float32)
    a_src = jnp.zeros((heads * _SEG, heads), jnp.float32)
    a_dst = jnp.zeros((heads * _SEG, heads), jnp.float32)
    for h in range(heads):
        w_cat = w_cat.at[:, h * _SEG:h * _SEG + d].set(W_heads[h])
        a_src = a_src.at[h * _SEG:h * _SEG + d, h].set(a_src_heads[h, :, 0])
        a_dst = a_dst.at[h * _SEG:h * _SEG + d, h].set(a_dst_heads[h, :, 0])
    return w_cat, a_src, a_dst


def kernel(inputs, bias_mat, training, W1, a_src1, b_src1, a_dst1, b_dst1,
           bias1, W2, a_src2, b_src2, a_dst2, b_dst2, bias2):
    n = inputs.shape[1]
    f_in = inputs.shape[2]
    heads1, _, h_dim = W1.shape
    c_dim = W2.shape[1]

    x = inputs.reshape(n, f_in)
    bias2d = bias_mat.reshape(n, n)
    rb_proj = 2000 if n % 2000 == 0 else n
    rb_attn = 200 if n % 200 == 0 else n

    w1_cat, a_src1_cat, a_dst1_cat = _pad_params(W1, a_src1, a_dst1, h_dim)
    fts1, f1_1, f2_1 = _project(x, w1_cat, a_src1_cat, a_dst1_cat,
                                b_src1.reshape(1, heads1),
                                b_dst1.reshape(1, heads1),
                                heads1, h_dim, rb_proj)
    rb_attn1 = 400 if n % 400 == 0 else rb_attn
    h1, mask8 = _attn_layer(bias2d, f1_1, f2_1.T, fts1,
                            bias1.reshape(1, heads1 * h_dim),
                            heads1, h_dim, elu=True, row_block=rb_attn1)

    w2_cat, a_src2_cat, a_dst2_cat = _pad_params(
        W2[None], a_src2[None], a_dst2[None], c_dim)
    fts2, f1_2, f2_2 = _project(h1, w2_cat, a_src2_cat, a_dst2_cat,
                                b_src2.reshape(1, 1), b_dst2.reshape(1, 1),
                                1, c_dim, rb_proj)
    rb_attn2 = 400 if n % 400 == 0 else rb_attn
    out, _ = _attn_layer(mask8, f1_2, f2_2.T, fts2, bias2.reshape(1, c_dim),
                         1, c_dim, elu=False, row_block=rb_attn2,
                         from_mask=True)
    return out.reshape(1, n, c_dim)


# speedup vs baseline: 1.3476x; 1.0093x over previous
import functools

import jax
import jax.numpy as jnp
from jax.experimental import pallas as pl

_SEG = 128


def _proj_body(x_ref, w_ref, asrc_ref, adst_ref, bs_ref, bd_ref,
               fts_ref, f1_ref, f2_ref, *, heads, d):
    fts = jnp.dot(x_ref[...], w_ref[...], preferred_element_type=jnp.float32)
    f1 = jnp.dot(fts, asrc_ref[...],
                 preferred_element_type=jnp.float32) + bs_ref[...]
    f2 = jnp.dot(fts, adst_ref[...],
                 preferred_element_type=jnp.float32) + bd_ref[...]
    f1_ref[...] = jnp.exp(-0.8 * f1).astype(jnp.bfloat16)
    f2_ref[...] = jnp.exp(0.8 * f2).astype(jnp.bfloat16)
    rb = fts.shape[0]
    vp = jnp.exp(0.2 * f2)
    vp_full = jnp.concatenate(
        [jnp.broadcast_to(vp[:, h:h + 1], (rb, _SEG)) for h in range(heads)],
        axis=1)
    fts_ref[...] = (fts * vp_full).astype(jnp.bfloat16)
    for h in range(heads):
        fts_ref[:, h * _SEG + d:h * _SEG + d + 1] = \
            vp[:, h:h + 1].astype(jnp.bfloat16)


def _project(x, w_cat, a_src_cat, a_dst_cat, b_src_row, b_dst_row,
             heads, d, row_block):
    n, fin = x.shape
    dtot = w_cat.shape[1]
    hh = a_src_cat.shape[1]
    grid = (n // row_block,)
    body = functools.partial(_proj_body, heads=heads, d=d)
    return pl.pallas_call(
        body,
        grid=grid,
        in_specs=[
            pl.BlockSpec((row_block, fin), lambda i: (i, 0)),
            pl.BlockSpec((fin, dtot), lambda i: (0, 0)),
            pl.BlockSpec((dtot, hh), lambda i: (0, 0)),
            pl.BlockSpec((dtot, hh), lambda i: (0, 0)),
            pl.BlockSpec((1, hh), lambda i: (0, 0)),
            pl.BlockSpec((1, hh), lambda i: (0, 0)),
        ],
        out_specs=[
            pl.BlockSpec((row_block, dtot), lambda i: (i, 0)),
            pl.BlockSpec((row_block, hh), lambda i: (i, 0)),
            pl.BlockSpec((row_block, hh), lambda i: (i, 0)),
        ],
        out_shape=[
            jax.ShapeDtypeStruct((n, dtot), jnp.bfloat16),
            jax.ShapeDtypeStruct((n, hh), jnp.bfloat16),
            jax.ShapeDtypeStruct((n, hh), jnp.bfloat16),
        ],
    )(x, w_cat, a_src_cat, a_dst_cat, b_src_row, b_dst_row)


def _attn_body(bias_ref, f1_ref, f2t_ref, fts_ref, bout_ref, out_ref,
               *maybe_mask, heads, d, elu, from_mask):
    if from_mask:
        eb = bias_ref[...].astype(jnp.bfloat16)
    else:
        bias_bf = bias_ref[...].astype(jnp.bfloat16)
        eb = jnp.clip(bias_bf + 1.0, 0.0, 1.0)
        maybe_mask[0][...] = eb.astype(jnp.int8)
    for h in range(heads):
        r = f1_ref[:, h][:, None]
        w = f2t_ref[h, :][None, :]
        e = jnp.maximum(w, r) * eb
        num = jnp.dot(e, fts_ref[:, h * _SEG:(h + 1) * _SEG],
                      preferred_element_type=jnp.float32)
        o = num[:, :d] / num[:, d:d + 1] + bout_ref[:, h * d:(h + 1) * d]
        if elu:
            o = jnp.where(o > 0, o, jnp.exp(jnp.minimum(o, 0.0)) - 1.0)
        out_ref[:, h * d:(h + 1) * d] = o


def _attn_layer(bias2d, f1, f2t, fts, b_out_row, heads, d, elu, row_block,
                from_mask=False):
    n = bias2d.shape[0]
    grid = (n // row_block,)
    body = functools.partial(_attn_body, heads=heads, d=d, elu=elu,
                             from_mask=from_mask)
    out_specs = [pl.BlockSpec((row_block, heads * d), lambda i: (i, 0)),
                 pl.BlockSpec((row_block, n), lambda i: (i, 0))]
    out_shape = [jax.ShapeDtypeStruct((n, heads * d), jnp.float32),
                 jax.ShapeDtypeStruct((n, n), jnp.int8)]
    if from_mask:
        out_specs, out_shape = out_specs[:1], out_shape[:1]
    res = pl.pallas_call(
        body,
        grid=grid,
        in_specs=[
            pl.BlockSpec((row_block, n), lambda i: (i, 0)),
            pl.BlockSpec((row_block, heads), lambda i: (i, 0)),
            pl.BlockSpec((heads, n), lambda i: (0, 0)),
            pl.BlockSpec((n, heads * _SEG), lambda i: (0, 0)),
            pl.BlockSpec((1, heads * d), lambda i: (0, 0)),
        ],
        out_specs=out_specs,
        out_shape=out_shape,
    )(bias2d, f1, f2t, fts, b_out_row)
    return (res[0], None) if from_mask else (res[0], res[1])


def _pad_params(W_heads, a_src_heads, a_dst_heads, d):
    heads, fin, _ = W_heads.shape
    w_cat = jnp.zeros((fin, heads * _SEG), jnp.float32)
    a_src = jnp.zeros((heads * _SEG, heads), jnp.float32)
    a_dst = jnp.zeros((heads * _SEG, heads), jnp.float32)
    for h in range(heads):
        w_cat = w_cat.at[:, h * _SEG:h * _SEG + d].set(W_heads[h])
        a_src = a_src.at[h * _SEG:h * _SEG + d, h].set(a_src_heads[h, :, 0])
        a_dst = a_dst.at[h * _SEG:h * _SEG + d, h].set(a_dst_heads[h, :, 0])
    return w_cat, a_src, a_dst


def kernel(inputs, bias_mat, training, W1, a_src1, b_src1, a_dst1, b_dst1,
           bias1, W2, a_src2, b_src2, a_dst2, b_dst2, bias2):
    n = inputs.shape[1]
    f_in = inputs.shape[2]
    heads1, _, h_dim = W1.shape
    c_dim = W2.shape[1]

    x = inputs.reshape(n, f_in)
    bias2d = bias_mat.reshape(n, n)
    rb_proj = 2000 if n % 2000 == 0 else n
    rb_attn = 200 if n % 200 == 0 else n

    w1_cat, a_src1_cat, a_dst1_cat = _pad_params(W1, a_src1, a_dst1, h_dim)
    fts1, f1_1, f2_1 = _project(x, w1_cat, a_src1_cat, a_dst1_cat,
                                b_src1.reshape(1, heads1),
                                b_dst1.reshape(1, heads1),
                                heads1, h_dim, rb_proj)
    rb_attn1 = 400 if n % 400 == 0 else rb_attn
    h1, mask8 = _attn_layer(bias2d, f1_1, f2_1.T, fts1,
                            bias1.reshape(1, heads1 * h_dim),
                            heads1, h_dim, elu=True, row_block=rb_attn1)

    w2_cat, a_src2_cat, a_dst2_cat = _pad_params(
        W2[None], a_src2[None], a_dst2[None], c_dim)
    fts2, f1_2, f2_2 = _project(h1, w2_cat, a_src2_cat, a_dst2_cat,
                                b_src2.reshape(1, 1), b_dst2.reshape(1, 1),
                                1, c_dim, rb_proj)
    rb_attn2 = 1000 if n % 1000 == 0 else rb_attn
    out, _ = _attn_layer(mask8, f1_2, f2_2.T, fts2, bias2.reshape(1, c_dim),
                         1, c_dim, elu=False, row_block=rb_attn2,
                         from_mask=True)
    return out.reshape(1, n, c_dim)


# speedup vs baseline: 1.3697x; 1.0164x over previous
import functools

import jax
import jax.numpy as jnp
from jax.experimental import pallas as pl

_SEG = 128


def _proj_body(x_ref, w_ref, asrc_ref, adst_ref, bs_ref, bd_ref,
               fts_ref, f1_ref, f2_ref, *, heads, d):
    fts = jnp.dot(x_ref[...], w_ref[...], preferred_element_type=jnp.float32)
    f1 = jnp.dot(fts, asrc_ref[...],
                 preferred_element_type=jnp.float32) + bs_ref[...]
    f2 = jnp.dot(fts, adst_ref[...],
                 preferred_element_type=jnp.float32) + bd_ref[...]
    f1_ref[...] = jnp.exp(-0.8 * f1).astype(jnp.bfloat16)
    f2_ref[...] = jnp.exp(0.8 * f2).astype(jnp.bfloat16)
    rb = fts.shape[0]
    vp = jnp.exp(0.2 * f2)
    vp_full = jnp.concatenate(
        [jnp.broadcast_to(vp[:, h:h + 1], (rb, _SEG)) for h in range(heads)],
        axis=1)
    fts_ref[...] = (fts * vp_full).astype(jnp.bfloat16)
    for h in range(heads):
        fts_ref[:, h * _SEG + d:h * _SEG + d + 1] = \
            vp[:, h:h + 1].astype(jnp.bfloat16)


def _project(x, w_cat, a_src_cat, a_dst_cat, b_src_row, b_dst_row,
             heads, d, row_block):
    n, fin = x.shape
    dtot = w_cat.shape[1]
    hh = a_src_cat.shape[1]
    grid = (n // row_block,)
    body = functools.partial(_proj_body, heads=heads, d=d)
    return pl.pallas_call(
        body,
        grid=grid,
        in_specs=[
            pl.BlockSpec((row_block, fin), lambda i: (i, 0)),
            pl.BlockSpec((fin, dtot), lambda i: (0, 0)),
            pl.BlockSpec((dtot, hh), lambda i: (0, 0)),
            pl.BlockSpec((dtot, hh), lambda i: (0, 0)),
            pl.BlockSpec((1, hh), lambda i: (0, 0)),
            pl.BlockSpec((1, hh), lambda i: (0, 0)),
        ],
        out_specs=[
            pl.BlockSpec((row_block, dtot), lambda i: (i, 0)),
            pl.BlockSpec((row_block, hh), lambda i: (i, 0)),
            pl.BlockSpec((row_block, hh), lambda i: (i, 0)),
        ],
        out_shape=[
            jax.ShapeDtypeStruct((n, dtot), jnp.bfloat16),
            jax.ShapeDtypeStruct((n, hh), jnp.bfloat16),
            jax.ShapeDtypeStruct((n, hh), jnp.bfloat16),
        ],
    )(x, w_cat, a_src_cat, a_dst_cat, b_src_row, b_dst_row)


def _attn_body(bias_ref, f1_ref, f2t_ref, fts_ref, bout_ref, out_ref,
               *maybe_mask, heads, d, elu, from_mask):
    if from_mask:
        eb = bias_ref[...].astype(jnp.bfloat16)
    else:
        bias_bf = bias_ref[...].astype(jnp.bfloat16)
        eb = jnp.clip(bias_bf + 1.0, 0.0, 1.0)
        maybe_mask[0][...] = eb.astype(jnp.int8)
    for h in range(heads):
        r = f1_ref[:, h][:, None]
        w = f2t_ref[h, :][None, :]
        e = jnp.maximum(w, r) * eb
        num = jnp.dot(e, fts_ref[:, h * _SEG:(h + 1) * _SEG],
                      preferred_element_type=jnp.float32)
        o = num[:, :d] / num[:, d:d + 1] + bout_ref[:, h * d:(h + 1) * d]
        if elu:
            o = jnp.where(o > 0, o, jnp.exp(jnp.minimum(o, 0.0)) - 1.0)
        out_ref[:, h * d:(h + 1) * d] = o


def _attn1_body(bias_ref, f1_ref, f2t_ref, fts_ref, bout_ref,
                w2_ref, as2_ref, ad2_ref, bs2_ref, bd2_ref,
                fts2_ref, r2_ref, w2o_ref, mask_ref, *, heads, d, d2):
    bias_bf = bias_ref[...].astype(jnp.bfloat16)
    eb = jnp.clip(bias_bf + 1.0, 0.0, 1.0)
    mask_ref[...] = eb.astype(jnp.int8)
    os = []
    for h in range(heads):
        r = f1_ref[:, h][:, None]
        w = f2t_ref[h, :][None, :]
        e = jnp.maximum(w, r) * eb
        num = jnp.dot(e, fts_ref[:, h * _SEG:(h + 1) * _SEG],
                      preferred_element_type=jnp.float32)
        o = num[:, :d] / num[:, d:d + 1] + bout_ref[:, h * d:(h + 1) * d]
        os.append(jnp.where(o > 0, o, jnp.exp(jnp.minimum(o, 0.0)) - 1.0))
    h1 = jnp.concatenate(os, axis=1)
    fts2 = jnp.dot(h1, w2_ref[...], preferred_element_type=jnp.float32)
    f1_2 = jnp.dot(fts2, as2_ref[...],
                   preferred_element_type=jnp.float32) + bs2_ref[...]
    f2_2 = jnp.dot(fts2, ad2_ref[...],
                   preferred_element_type=jnp.float32) + bd2_ref[...]
    r2_ref[...] = jnp.exp(-0.8 * f1_2).astype(jnp.bfloat16)
    w2o_ref[...] = jnp.exp(0.8 * f2_2).astype(jnp.bfloat16)
    vp2 = jnp.exp(0.2 * f2_2)
    fts2_ref[...] = (fts2 * vp2).astype(jnp.bfloat16)
    fts2_ref[:, d2:d2 + 1] = vp2.astype(jnp.bfloat16)


def _attn1_fused(bias2d, f1, f2t, fts, b_out_row,
                 w2c, as2c, ad2c, bs2, bd2, heads, d, d2, row_block):
    n = bias2d.shape[0]
    grid = (n // row_block,)
    body = functools.partial(_attn1_body, heads=heads, d=d, d2=d2)
    return pl.pallas_call(
        body,
        grid=grid,
        in_specs=[
            pl.BlockSpec((row_block, n), lambda i: (i, 0)),
            pl.BlockSpec((row_block, heads), lambda i: (i, 0)),
            pl.BlockSpec((heads, n), lambda i: (0, 0)),
            pl.BlockSpec((n, heads * _SEG), lambda i: (0, 0)),
            pl.BlockSpec((1, heads * d), lambda i: (0, 0)),
            pl.BlockSpec(w2c.shape, lambda i: (0, 0)),
            pl.BlockSpec(as2c.shape, lambda i: (0, 0)),
            pl.BlockSpec(ad2c.shape, lambda i: (0, 0)),
            pl.BlockSpec((1, 1), lambda i: (0, 0)),
            pl.BlockSpec((1, 1), lambda i: (0, 0)),
        ],
        out_specs=[
            pl.BlockSpec((row_block, _SEG), lambda i: (i, 0)),
            pl.BlockSpec((row_block, 1), lambda i: (i, 0)),
            pl.BlockSpec((row_block, 1), lambda i: (i, 0)),
            pl.BlockSpec((row_block, n), lambda i: (i, 0)),
        ],
        out_shape=[
            jax.ShapeDtypeStruct((n, _SEG), jnp.bfloat16),
            jax.ShapeDtypeStruct((n, 1), jnp.bfloat16),
            jax.ShapeDtypeStruct((n, 1), jnp.bfloat16),
            jax.ShapeDtypeStruct((n, n), jnp.int8),
        ],
    )(bias2d, f1, f2t, fts, b_out_row, w2c, as2c, ad2c, bs2, bd2)


def _attn_layer(bias2d, f1, f2t, fts, b_out_row, heads, d, elu, row_block,
                from_mask=False):
    n = bias2d.shape[0]
    grid = (n // row_block,)
    body = functools.partial(_attn_body, heads=heads, d=d, elu=elu,
                             from_mask=from_mask)
    out_specs = [pl.BlockSpec((row_block, heads * d), lambda i: (i, 0)),
                 pl.BlockSpec((row_block, n), lambda i: (i, 0))]
    out_shape = [jax.ShapeDtypeStruct((n, heads * d), jnp.float32),
                 jax.ShapeDtypeStruct((n, n), jnp.int8)]
    if from_mask:
        out_specs, out_shape = out_specs[:1], out_shape[:1]
    res = pl.pallas_call(
        body,
        grid=grid,
        in_specs=[
            pl.BlockSpec((row_block, n), lambda i: (i, 0)),
            pl.BlockSpec((row_block, heads), lambda i: (i, 0)),
            pl.BlockSpec((heads, n), lambda i: (0, 0)),
            pl.BlockSpec((n, heads * _SEG), lambda i: (0, 0)),
            pl.BlockSpec((1, heads * d), lambda i: (0, 0)),
        ],
        out_specs=out_specs,
        out_shape=out_shape,
    )(bias2d, f1, f2t, fts, b_out_row)
    return (res[0], None) if from_mask else (res[0], res[1])


def _pad_params(W_heads, a_src_heads, a_dst_heads, d):
    heads, fin, _ = W_heads.shape
    w_cat = jnp.zeros((fin, heads * _SEG), jnp.float32)
    a_src = jnp.zeros((heads * _SEG, heads), jnp.float32)
    a_dst = jnp.zeros((heads * _SEG, heads), jnp.float32)
    for h in range(heads):
        w_cat = w_cat.at[:, h * _SEG:h * _SEG + d].set(W_heads[h])
        a_src = a_src.at[h * _SEG:h * _SEG + d, h].set(a_src_heads[h, :, 0])
        a_dst = a_dst.at[h * _SEG:h * _SEG + d, h].set(a_dst_heads[h, :, 0])
    return w_cat, a_src, a_dst


def kernel(inputs, bias_mat, training, W1, a_src1, b_src1, a_dst1, b_dst1,
           bias1, W2, a_src2, b_src2, a_dst2, b_dst2, bias2):
    n = inputs.shape[1]
    f_in = inputs.shape[2]
    heads1, _, h_dim = W1.shape
    c_dim = W2.shape[1]

    x = inputs.reshape(n, f_in)
    bias2d = bias_mat.reshape(n, n)
    rb_proj = 2000 if n % 2000 == 0 else n
    rb_attn = 200 if n % 200 == 0 else n

    w1_cat, a_src1_cat, a_dst1_cat = _pad_params(W1, a_src1, a_dst1, h_dim)
    fts1, f1_1, f2_1 = _project(x, w1_cat, a_src1_cat, a_dst1_cat,
                                b_src1.reshape(1, heads1),
                                b_dst1.reshape(1, heads1),
                                heads1, h_dim, rb_proj)
    w2_cat, a_src2_cat, a_dst2_cat = _pad_params(
        W2[None], a_src2[None], a_dst2[None], c_dim)
    rb_attn1 = 400 if n % 400 == 0 else rb_attn
    fts2, r2, w2o, mask8 = _attn1_fused(
        bias2d, f1_1, f2_1.T, fts1, bias1.reshape(1, heads1 * h_dim),
        w2_cat, a_src2_cat, a_dst2_cat,
        b_src2.reshape(1, 1), b_dst2.reshape(1, 1),
        heads1, h_dim, c_dim, rb_attn1)

    rb_attn2 = 1000 if n % 1000 == 0 else rb_attn
    out, _ = _attn_layer(mask8, r2, w2o.T, fts2, bias2.reshape(1, c_dim),
                         1, c_dim, elu=False, row_block=rb_attn2,
                         from_mask=True)
    return out.reshape(1, n, c_dim)
